# baseline (speedup 1.0000x reference)
import functools

import jax
import jax.numpy as jnp
from jax import lax
from jax.experimental import pallas as pl
from jax.experimental.pallas import tpu as pltpu
from jax.experimental.pallas import tpu_sc as plsc

_NC, _NS = 2, 16
_NW = _NC * _NS
_NBUF = 2


def _sc_gather_body(idx_hbm, table_hbm, out_hbm, idx_v, bufs, gsems, ssems,
                    *, rows_per_w, chunk):
    wid = lax.axis_index("s") * _NC + lax.axis_index("c")
    base = wid * rows_per_w
    pltpu.sync_copy(idx_hbm.at[wid], idx_v)
    n_iter = rows_per_w // chunk

    for b in range(_NBUF):
        pltpu.async_copy(table_hbm.at[idx_v.at[b]], bufs[b], gsems[b])

    @pl.loop(0, n_iter, step=_NBUF)
    def _(g):
        for b in range(_NBUF):
            i = g + b
            pltpu.make_async_copy(
                table_hbm.at[idx_v.at[i]], bufs[b], gsems[b]).wait()
            out_slice = out_hbm.at[pl.ds(base + i * chunk, chunk)]
            pltpu.async_copy(bufs[b], out_slice, ssems[b])
            pltpu.make_async_copy(bufs[b], out_slice, ssems[b]).wait()
            nxt = i + _NBUF

            @pl.when(nxt < n_iter)
            def _():
                pltpu.async_copy(
                    table_hbm.at[idx_v.at[nxt]], bufs[b], gsems[b])


def _sc_gather(idx_flat, table):
    n_rows = idx_flat.shape[0]
    C = table.shape[1]
    rows_per_w = n_rows // _NW
    chunk = 4
    idx3 = idx_flat.reshape(_NW, rows_per_w // chunk, chunk)
    mesh = plsc.VectorSubcoreMesh(
        core_axis_name="c", subcore_axis_name="s",
        num_cores=_NC, num_subcores=_NS)
    body = functools.partial(_sc_gather_body, rows_per_w=rows_per_w,
                             chunk=chunk)

    def wrapped(idx_hbm, table_hbm, out_hbm, *scratch):
        bufs = scratch[:_NBUF]
        gsems = scratch[_NBUF:2 * _NBUF]
        ssems = scratch[2 * _NBUF:3 * _NBUF]
        body(idx_hbm, table_hbm, out_hbm, scratch[3 * _NBUF], bufs, gsems,
             ssems)

    return pl.kernel(
        wrapped,
        out_type=jax.ShapeDtypeStruct((n_rows, C), jnp.float32),
        mesh=mesh,
        scratch_types=(
            [pltpu.VMEM((chunk, C), jnp.float32)] * _NBUF
            + [pltpu.SemaphoreType.DMA] * (2 * _NBUF)
            + [pltpu.VMEM((rows_per_w // chunk, chunk), jnp.int32)]
        ),
    )(idx3, table)


def _lse_body(rows_ref, lse_ref):
    rows = rows_ref[...]
    m = jnp.max(rows, axis=1, keepdims=True)
    e = jnp.exp(rows - m)
    s = jnp.sum(e, axis=1, keepdims=True)
    lse_ref[...] = m + jnp.log(s)


def _tc_lse(table):
    V, C = table.shape
    R = 128
    lse = pl.pallas_call(
        _lse_body,
        grid=(V // R,),
        in_specs=[pl.BlockSpec((R, C), lambda i: (i, 0))],
        out_specs=pl.BlockSpec((R, 1), lambda i: (i, 0)),
        out_shape=jax.ShapeDtypeStruct((V, 1), jnp.float32),
    )(table)
    return lse.reshape(V)


def _nll_body(idxe_hbm, fidx_hbm, lse_hbm, tflat_hbm, out_hbm,
              idxe_v, fidx_v, lse_buf, tv_buf, acc_vm, sem, *, rows_per_w):
    wid = lax.axis_index("s") * _NC + lax.axis_index("c")
    n_dma = rows_per_w // 128
    pltpu.sync_copy(idxe_hbm.at[wid], idxe_v)
    pltpu.sync_copy(fidx_hbm.at[wid], fidx_v)
    for j in range(n_dma):
        pltpu.async_copy(lse_hbm.at[idxe_v.at[j]],
                         lse_buf.at[pl.ds(j * 128, 128)], sem)
        pltpu.async_copy(tflat_hbm.at[fidx_v.at[j]],
                         tv_buf.at[pl.ds(j * 128, 128)], sem)
    for j in range(n_dma):
        pltpu.make_async_copy(lse_hbm.at[idxe_v.at[j]],
                              lse_buf.at[pl.ds(j * 128, 128)], sem).wait()
        pltpu.make_async_copy(tflat_hbm.at[fidx_v.at[j]],
                              tv_buf.at[pl.ds(j * 128, 128)], sem).wait()
    acc = jnp.zeros((16,), jnp.float32)
    for k in range(rows_per_w // 16):
        acc = acc + (lse_buf[pl.ds(k * 16, 16)] - tv_buf[pl.ds(k * 16, 16)])
    acc_vm[...] = acc
    pltpu.sync_copy(acc_vm, out_hbm.at[wid])


def _sc_nll_partials(idx_flat, fidx_flat, lse, table_flat):
    n_rows = idx_flat.shape[0]
    rows_per_w = n_rows // _NW
    n_dma = rows_per_w // 128
    idx3 = idx_flat.reshape(_NW, n_dma, 128)
    fidx3 = fidx_flat.reshape(_NW, n_dma, 128)
    mesh = plsc.VectorSubcoreMesh(
        core_axis_name="c", subcore_axis_name="s",
        num_cores=_NC, num_subcores=_NS)
    return pl.kernel(
        functools.partial(_nll_body, rows_per_w=rows_per_w),
        out_type=jax.ShapeDtypeStruct((_NW, 16), jnp.float32),
        mesh=mesh,
        scratch_types=(
            pltpu.VMEM((n_dma, 128), jnp.int32),
            pltpu.VMEM((n_dma, 128), jnp.int32),
            pltpu.VMEM((rows_per_w,), jnp.float32),
            pltpu.VMEM((rows_per_w,), jnp.float32),
            pltpu.VMEM((16,), jnp.float32),
            pltpu.SemaphoreType.DMA,
        ),
    )(idx3, fidx3, lse, table_flat)


def _mean_body(part_ref, loss_ref, *, n_rows):
    loss_ref[...] = jnp.full(
        (1, 1), jnp.sum(part_ref[...]) / jnp.float32(n_rows), jnp.float32)


def kernel(idx, targets, table):
    B, T = idx.shape
    N = B * T
    V, C = table.shape
    idx_flat = idx.reshape(N).astype(jnp.int32)
    tgt_flat = targets.reshape(N).astype(jnp.int32)
    fidx_flat = idx_flat * jnp.int32(C) + tgt_flat

    logits = _sc_gather(idx_flat, table)
    lse = _tc_lse(table)
    partials = _sc_nll_partials(idx_flat, fidx_flat, lse,
                                table.reshape(V * C))
    loss = pl.pallas_call(
        functools.partial(_mean_body, n_rows=N),
        out_shape=jax.ShapeDtypeStruct((1, 1), jnp.float32),
    )(partials)
    return (logits, loss[0, 0])


# speedup vs baseline: 1.3024x; 1.3024x over previous
import functools

import jax
import jax.numpy as jnp
from jax import lax
from jax.experimental import pallas as pl
from jax.experimental.pallas import tpu as pltpu
from jax.experimental.pallas import tpu_sc as plsc

_NC, _NS = 2, 16
_NW = _NC * _NS
_NBUF = 2


def _sc_gather_body(idx_hbm, tgt16_hbm, table_hbm, out_hbm, tv_hbm,
                    idx_v, tgt16_v, tvacc_vm, bufs, gsems, ssems,
                    *, rows_per_w, chunk):
    wid = lax.axis_index("s") * _NC + lax.axis_index("c")
    base = wid * rows_per_w
    pltpu.sync_copy(idx_hbm.at[wid], idx_v)
    pltpu.sync_copy(tgt16_hbm.at[wid], tgt16_v)
    n_iter = rows_per_w // chunk
    lane16 = lax.iota(jnp.int32, 16)
    row_ids = lax.rem(lane16, jnp.int32(chunk))
    lane_on = lane16 < chunk

    for b in range(_NBUF):
        pltpu.async_copy(table_hbm.at[idx_v.at[b]], bufs[b], gsems[b])

    def _loop_body(g, acc):
        for b in range(_NBUF):
            i = g + b
            pltpu.make_async_copy(
                table_hbm.at[idx_v.at[i]], bufs[b], gsems[b]).wait()
            col_ids = tgt16_v[i, :]
            vals = plsc.load_gather(bufs[b], [row_ids, col_ids])
            acc = acc + jnp.where(lane_on, vals, 0.0)
            out_slice = out_hbm.at[pl.ds(base + i * chunk, chunk)]
            pltpu.async_copy(bufs[b], out_slice, ssems[b])
            pltpu.make_async_copy(bufs[b], out_slice, ssems[b]).wait()
            nxt = i + _NBUF

            @pl.when(nxt < n_iter)
            def _():
                pltpu.async_copy(
                    table_hbm.at[idx_v.at[nxt]], bufs[b], gsems[b])
        return acc

    final_acc = pl.loop(0, n_iter, step=_NBUF,
                        init_carry=jnp.zeros((16,), jnp.float32))(_loop_body)
    tvacc_vm[...] = final_acc
    pltpu.sync_copy(tvacc_vm, tv_hbm.at[wid])


def _sc_gather(idx_flat, tgt16, table):
    n_rows = idx_flat.shape[0]
    C = table.shape[1]
    rows_per_w = n_rows // _NW
    chunk = 4
    n_iter = rows_per_w // chunk
    idx3 = idx_flat.reshape(_NW, n_iter, chunk)
    mesh = plsc.VectorSubcoreMesh(
        core_axis_name="c", subcore_axis_name="s",
        num_cores=_NC, num_subcores=_NS)
    body = functools.partial(_sc_gather_body, rows_per_w=rows_per_w,
                             chunk=chunk)

    def wrapped(idx_hbm, tgt16_hbm, table_hbm, out_hbm, tv_hbm, *scratch):
        bufs = scratch[:_NBUF]
        gsems = scratch[_NBUF:2 * _NBUF]
        ssems = scratch[2 * _NBUF:3 * _NBUF]
        body(idx_hbm, tgt16_hbm, table_hbm, out_hbm, tv_hbm,
             scratch[3 * _NBUF], scratch[3 * _NBUF + 1],
             scratch[3 * _NBUF + 2], bufs, gsems, ssems)

    return pl.kernel(
        wrapped,
        out_type=(jax.ShapeDtypeStruct((n_rows, C), jnp.float32),
                  jax.ShapeDtypeStruct((_NW, 16), jnp.float32)),
        mesh=mesh,
        compiler_params=pltpu.CompilerParams(needs_layout_passes=False),
        scratch_types=(
            [pltpu.VMEM((chunk, C), jnp.float32)] * _NBUF
            + [pltpu.SemaphoreType.DMA] * (2 * _NBUF)
            + [pltpu.VMEM((n_iter, chunk), jnp.int32),
               pltpu.VMEM((n_iter, 16), jnp.int32),
               pltpu.VMEM((16,), jnp.float32)]
        ),
    )(idx3, tgt16, table)


def _lse_body(rows_ref, lse_ref):
    rows = rows_ref[...]
    m = jnp.max(rows, axis=1, keepdims=True)
    e = jnp.exp(rows - m)
    s = jnp.sum(e, axis=1, keepdims=True)
    lse_ref[...] = m + jnp.log(s)


def _tc_lse(table):
    V, C = table.shape
    R = 128
    lse = pl.pallas_call(
        _lse_body,
        grid=(V // R,),
        in_specs=[pl.BlockSpec((R, C), lambda i: (i, 0))],
        out_specs=pl.BlockSpec((R, 1), lambda i: (i, 0)),
        out_shape=jax.ShapeDtypeStruct((V, 1), jnp.float32),
    )(table)
    return lse.reshape(V)


def _nll_body(idxe_hbm, lse_hbm, out_hbm, idxe_v, lse_buf, acc_vm, sem,
              *, rows_per_w):
    wid = lax.axis_index("s") * _NC + lax.axis_index("c")
    n_dma = rows_per_w // 128
    pltpu.sync_copy(idxe_hbm.at[wid], idxe_v)
    for j in range(n_dma):
        pltpu.async_copy(lse_hbm.at[idxe_v.at[j]],
                         lse_buf.at[pl.ds(j * 128, 128)], sem)
    for j in range(n_dma):
        pltpu.make_async_copy(lse_hbm.at[idxe_v.at[j]],
                              lse_buf.at[pl.ds(j * 128, 128)], sem).wait()
    acc = jnp.zeros((16,), jnp.float32)
    for k in range(rows_per_w // 16):
        acc = acc + lse_buf[pl.ds(k * 16, 16)]
    acc_vm[...] = acc
    pltpu.sync_copy(acc_vm, out_hbm.at[wid])


def _sc_lse_partials(idx_flat, lse):
    n_rows = idx_flat.shape[0]
    rows_per_w = n_rows // _NW
    n_dma = rows_per_w // 128
    idx3 = idx_flat.reshape(_NW, n_dma, 128)
    mesh = plsc.VectorSubcoreMesh(
        core_axis_name="c", subcore_axis_name="s",
        num_cores=_NC, num_subcores=_NS)
    return pl.kernel(
        functools.partial(_nll_body, rows_per_w=rows_per_w),
        out_type=jax.ShapeDtypeStruct((_NW, 16), jnp.float32),
        mesh=mesh,
        scratch_types=(
            pltpu.VMEM((n_dma, 128), jnp.int32),
            pltpu.VMEM((rows_per_w,), jnp.float32),
            pltpu.VMEM((16,), jnp.float32),
            pltpu.SemaphoreType.DMA,
        ),
    )(idx3, lse)


def _mean_body(lse_part_ref, tv_part_ref, loss_ref, *, n_rows):
    nll_sum = jnp.sum(lse_part_ref[...]) - jnp.sum(tv_part_ref[...])
    loss_ref[...] = jnp.full((1, 1), nll_sum / jnp.float32(n_rows),
                             jnp.float32)


def kernel(idx, targets, table):
    B, T = idx.shape
    N = B * T
    V, C = table.shape
    chunk = 4
    idx_flat = idx.reshape(N).astype(jnp.int32)
    tgt_flat = targets.reshape(N).astype(jnp.int32)
    tgt16 = jnp.pad(tgt_flat.reshape(_NW, N // (_NW * chunk), 1, chunk),
                    ((0, 0), (0, 0), (0, 0), (0, 16 - chunk))
                    ).reshape(_NW, N // (_NW * chunk), 16)

    logits, tv_partials = _sc_gather(idx_flat, tgt16, table)
    lse = _tc_lse(table)
    lse_partials = _sc_lse_partials(idx_flat, lse)
    loss = pl.pallas_call(
        functools.partial(_mean_body, n_rows=N),
        out_shape=jax.ShapeDtypeStruct((1, 1), jnp.float32),
    )(lse_partials, tv_partials)
    return (logits, loss[0, 0])


# speedup vs baseline: 1.3560x; 1.0411x over previous
import functools

import jax
import jax.numpy as jnp
from jax import lax
from jax.experimental import pallas as pl
from jax.experimental.pallas import tpu as pltpu
from jax.experimental.pallas import tpu_sc as plsc

_NC, _NS = 2, 16
_NW = _NC * _NS
_NBUF = 3
_CHUNK = 2


def _sc_gather_body(idx_hbm, tgt16_hbm, table_hbm, out_hbm, tv_hbm,
                    idx_v, tgt16_v, tvacc_vm, bufs, gsems, ssems,
                    *, rows_per_w, chunk):
    wid = lax.axis_index("s") * _NC + lax.axis_index("c")
    base = wid * rows_per_w
    pltpu.sync_copy(idx_hbm.at[wid], idx_v)
    pltpu.sync_copy(tgt16_hbm.at[wid], tgt16_v)
    n_iter = rows_per_w // chunk
    lane16 = lax.iota(jnp.int32, 16)
    row_ids = lax.rem(lane16, jnp.int32(chunk))
    lane_on = lane16 < chunk

    for b in range(_NBUF):
        pltpu.async_copy(table_hbm.at[idx_v.at[b]], bufs[b], gsems[b])

    def _step(i, b, acc):
        pltpu.make_async_copy(
            table_hbm.at[idx_v.at[i]], bufs[b], gsems[b]).wait()
        col_ids = tgt16_v[i, :]
        vals = plsc.load_gather(bufs[b], [row_ids, col_ids])
        acc = acc + jnp.where(lane_on, vals, 0.0)
        out_slice = out_hbm.at[pl.ds(base + i * chunk, chunk)]
        pltpu.async_copy(bufs[b], out_slice, ssems[b])
        pltpu.make_async_copy(bufs[b], out_slice, ssems[b]).wait()
        nxt = i + _NBUF
        if not (isinstance(nxt, int) and nxt >= n_iter):

            @pl.when(nxt < n_iter)
            def _():
                pltpu.async_copy(
                    table_hbm.at[idx_v.at[nxt]], bufs[b], gsems[b])
        return acc

    n_main = (n_iter // _NBUF) * _NBUF

    def _loop_body(g, acc):
        for b in range(_NBUF):
            acc = _step(g + b, b, acc)
        return acc

    final_acc = pl.loop(0, n_main, step=_NBUF,
                        init_carry=jnp.zeros((16,), jnp.float32))(_loop_body)
    for t in range(n_main, n_iter):
        final_acc = _step(t, t % _NBUF, final_acc)
    tvacc_vm[...] = final_acc
    pltpu.sync_copy(tvacc_vm, tv_hbm.at[wid])


def _sc_gather(idx_flat, tgt16, table):
    n_rows = idx_flat.shape[0]
    C = table.shape[1]
    rows_per_w = n_rows // _NW
    chunk = _CHUNK
    n_iter = rows_per_w // chunk
    idx3 = idx_flat.reshape(_NW, n_iter, chunk)
    mesh = plsc.VectorSubcoreMesh(
        core_axis_name="c", subcore_axis_name="s",
        num_cores=_NC, num_subcores=_NS)
    body = functools.partial(_sc_gather_body, rows_per_w=rows_per_w,
                             chunk=chunk)

    def wrapped(idx_hbm, tgt16_hbm, table_hbm, out_hbm, tv_hbm, *scratch):
        bufs = scratch[:_NBUF]
        gsems = scratch[_NBUF:2 * _NBUF]
        ssems = scratch[2 * _NBUF:3 * _NBUF]
        body(idx_hbm, tgt16_hbm, table_hbm, out_hbm, tv_hbm,
             scratch[3 * _NBUF], scratch[3 * _NBUF + 1],
             scratch[3 * _NBUF + 2], bufs, gsems, ssems)

    return pl.kernel(
        wrapped,
        out_type=(jax.ShapeDtypeStruct((n_rows, C), jnp.float32),
                  jax.ShapeDtypeStruct((_NW, 16), jnp.float32)),
        mesh=mesh,
        compiler_params=pltpu.CompilerParams(needs_layout_passes=False),
        scratch_types=(
            [pltpu.VMEM((chunk, C), jnp.float32)] * _NBUF
            + [pltpu.SemaphoreType.DMA] * (2 * _NBUF)
            + [pltpu.VMEM((n_iter, chunk), jnp.int32),
               pltpu.VMEM((n_iter, 16), jnp.int32),
               pltpu.VMEM((16,), jnp.float32)]
        ),
    )(idx3, tgt16, table)


def _lse_body(rows_ref, lse_ref):
    rows = rows_ref[...]
    m = jnp.max(rows, axis=1, keepdims=True)
    e = jnp.exp(rows - m)
    s = jnp.sum(e, axis=1, keepdims=True)
    lse_ref[...] = m + jnp.log(s)


def _tc_lse(table):
    V, C = table.shape
    R = 128
    lse = pl.pallas_call(
        _lse_body,
        grid=(V // R,),
        in_specs=[pl.BlockSpec((R, C), lambda i: (i, 0))],
        out_specs=pl.BlockSpec((R, 1), lambda i: (i, 0)),
        out_shape=jax.ShapeDtypeStruct((V, 1), jnp.float32),
    )(table)
    return lse.reshape(V)


def _nll_body(idxe_hbm, lse_hbm, out_hbm, idxe_v, lse_buf, acc_vm, sem,
              *, rows_per_w):
    wid = lax.axis_index("s") * _NC + lax.axis_index("c")
    n_dma = rows_per_w // 128
    pltpu.sync_copy(idxe_hbm.at[wid], idxe_v)
    for j in range(n_dma):
        pltpu.async_copy(lse_hbm.at[idxe_v.at[j]],
                         lse_buf.at[pl.ds(j * 128, 128)], sem)
    for j in range(n_dma):
        pltpu.make_async_copy(lse_hbm.at[idxe_v.at[j]],
                              lse_buf.at[pl.ds(j * 128, 128)], sem).wait()
    acc = jnp.zeros((16,), jnp.float32)
    for k in range(rows_per_w // 16):
        acc = acc + lse_buf[pl.ds(k * 16, 16)]
    acc_vm[...] = acc
    pltpu.sync_copy(acc_vm, out_hbm.at[wid])


def _sc_lse_partials(idx_flat, lse):
    n_rows = idx_flat.shape[0]
    rows_per_w = n_rows // _NW
    n_dma = rows_per_w // 128
    idx3 = idx_flat.reshape(_NW, n_dma, 128)
    mesh = plsc.VectorSubcoreMesh(
        core_axis_name="c", subcore_axis_name="s",
        num_cores=_NC, num_subcores=_NS)
    return pl.kernel(
        functools.partial(_nll_body, rows_per_w=rows_per_w),
        out_type=jax.ShapeDtypeStruct((_NW, 16), jnp.float32),
        mesh=mesh,
        scratch_types=(
            pltpu.VMEM((n_dma, 128), jnp.int32),
            pltpu.VMEM((rows_per_w,), jnp.float32),
            pltpu.VMEM((16,), jnp.float32),
            pltpu.SemaphoreType.DMA,
        ),
    )(idx3, lse)


def _mean_body(lse_part_ref, tv_part_ref, loss_ref, *, n_rows):
    nll_sum = jnp.sum(lse_part_ref[...]) - jnp.sum(tv_part_ref[...])
    loss_ref[...] = jnp.full((1, 1), nll_sum / jnp.float32(n_rows),
                             jnp.float32)


def kernel(idx, targets, table):
    B, T = idx.shape
    N = B * T
    V, C = table.shape
    chunk = _CHUNK
    idx_flat = idx.reshape(N).astype(jnp.int32)
    tgt_flat = targets.reshape(N).astype(jnp.int32)
    tgt16 = jnp.pad(tgt_flat.reshape(_NW, N // (_NW * chunk), 1, chunk),
                    ((0, 0), (0, 0), (0, 0), (0, 16 - chunk))
                    ).reshape(_NW, N // (_NW * chunk), 16)

    logits, tv_partials = _sc_gather(idx_flat, tgt16, table)
    lse = _tc_lse(table)
    lse_partials = _sc_lse_partials(idx_flat, lse)
    loss = pl.pallas_call(
        functools.partial(_mean_body, n_rows=N),
        out_shape=jax.ShapeDtypeStruct((1, 1), jnp.float32),
    )(lse_partials, tv_partials)
    return (logits, loss[0, 0])


# speedup vs baseline: 1.3594x; 1.0025x over previous
import functools

import jax
import jax.numpy as jnp
from jax import lax
from jax.experimental import pallas as pl
from jax.experimental.pallas import tpu as pltpu
from jax.experimental.pallas import tpu_sc as plsc

_NC, _NS = 2, 16
_NW = _NC * _NS
_NBUF = 4
_CHUNK = 2


def _sc_gather_body(idx_hbm, tgt16_hbm, table_hbm, out_hbm, tv_hbm,
                    idx_v, tgt16_v, tvacc_vm, bufs, gsems, ssems,
                    *, rows_per_w, chunk):
    wid = lax.axis_index("s") * _NC + lax.axis_index("c")
    base = wid * rows_per_w
    pltpu.sync_copy(idx_hbm.at[wid], idx_v)
    pltpu.sync_copy(tgt16_hbm.at[wid], tgt16_v)
    n_iter = rows_per_w // chunk
    lane16 = lax.iota(jnp.int32, 16)

    for b in range(_NBUF):
        pltpu.async_copy(table_hbm.at[idx_v.at[b]], bufs[b], gsems[b])

    def _step(i, b, acc):
        pltpu.make_async_copy(
            table_hbm.at[idx_v.at[i]], bufs[b], gsems[b]).wait()
        col_ids = tgt16_v[i // 2, :]
        shifted = lane16 - (i % 2) * chunk
        lane_on = (shifted >= 0) & (shifted < chunk)
        row_ids = jnp.clip(shifted, 0, chunk - 1)
        vals = plsc.load_gather(bufs[b], [row_ids, col_ids])
        acc = acc + jnp.where(lane_on, vals, 0.0)
        out_slice = out_hbm.at[pl.ds(base + i * chunk, chunk)]
        pltpu.async_copy(bufs[b], out_slice, ssems[b])
        pltpu.make_async_copy(bufs[b], out_slice, ssems[b]).wait()
        nxt = i + _NBUF
        if not (isinstance(nxt, int) and nxt >= n_iter):

            @pl.when(nxt < n_iter)
            def _():
                pltpu.async_copy(
                    table_hbm.at[idx_v.at[nxt]], bufs[b], gsems[b])
        return acc

    n_main = (n_iter // _NBUF) * _NBUF

    def _loop_body(g, acc):
        for b in range(_NBUF):
            acc = _step(g + b, b, acc)
        return acc

    final_acc = pl.loop(0, n_main, step=_NBUF,
                        init_carry=jnp.zeros((16,), jnp.float32))(_loop_body)
    for t in range(n_main, n_iter):
        final_acc = _step(t, t % _NBUF, final_acc)
    tvacc_vm[...] = final_acc
    pltpu.sync_copy(tvacc_vm, tv_hbm.at[wid])


def _sc_gather(idx_flat, tgt16, table):
    n_rows = idx_flat.shape[0]
    C = table.shape[1]
    rows_per_w = n_rows // _NW
    chunk = _CHUNK
    n_iter = rows_per_w // chunk
    idx3 = idx_flat.reshape(_NW, n_iter, chunk)
    mesh = plsc.VectorSubcoreMesh(
        core_axis_name="c", subcore_axis_name="s",
        num_cores=_NC, num_subcores=_NS)
    body = functools.partial(_sc_gather_body, rows_per_w=rows_per_w,
                             chunk=chunk)

    def wrapped(idx_hbm, tgt16_hbm, table_hbm, out_hbm, tv_hbm, *scratch):
        bufs = scratch[:_NBUF]
        gsems = scratch[_NBUF:2 * _NBUF]
        ssems = scratch[2 * _NBUF:3 * _NBUF]
        body(idx_hbm, tgt16_hbm, table_hbm, out_hbm, tv_hbm,
             scratch[3 * _NBUF], scratch[3 * _NBUF + 1],
             scratch[3 * _NBUF + 2], bufs, gsems, ssems)

    return pl.kernel(
        wrapped,
        out_type=(jax.ShapeDtypeStruct((n_rows, C), jnp.float32),
                  jax.ShapeDtypeStruct((_NW, 16), jnp.float32)),
        mesh=mesh,
        compiler_params=pltpu.CompilerParams(needs_layout_passes=False),
        scratch_types=(
            [pltpu.VMEM((chunk, C), jnp.float32)] * _NBUF
            + [pltpu.SemaphoreType.DMA] * (2 * _NBUF)
            + [pltpu.VMEM((n_iter, chunk), jnp.int32),
               pltpu.VMEM((n_iter // 2, 16), jnp.int32),
               pltpu.VMEM((16,), jnp.float32)]
        ),
    )(idx3, tgt16, table)


def _lse_body(rows_ref, lse_ref):
    rows = rows_ref[...]
    m = jnp.max(rows, axis=1, keepdims=True)
    e = jnp.exp(rows - m)
    s = jnp.sum(e, axis=1, keepdims=True)
    lse_ref[...] = m + jnp.log(s)


def _tc_lse(table):
    V, C = table.shape
    R = 256
    lse = pl.pallas_call(
        _lse_body,
        grid=(V // R,),
        in_specs=[pl.BlockSpec((R, C), lambda i: (i, 0))],
        out_specs=pl.BlockSpec((R, 1), lambda i: (i, 0)),
        out_shape=jax.ShapeDtypeStruct((V, 1), jnp.float32),
    )(table)
    return lse.reshape(V)


def _nll_body(idxe_hbm, lse_hbm, out_hbm, idxe_v, lse_buf, acc_vm, sem,
              *, rows_per_w):
    wid = lax.axis_index("s") * _NC + lax.axis_index("c")
    n_dma = rows_per_w // 128
    pltpu.sync_copy(idxe_hbm.at[wid], idxe_v)
    for j in range(n_dma):
        pltpu.async_copy(lse_hbm.at[idxe_v.at[j]],
                         lse_buf.at[pl.ds(j * 128, 128)], sem)
    for j in range(n_dma):
        pltpu.make_async_copy(lse_hbm.at[idxe_v.at[j]],
                              lse_buf.at[pl.ds(j * 128, 128)], sem).wait()
    acc = jnp.zeros((16,), jnp.float32)
    for k in range(rows_per_w // 16):
        acc = acc + lse_buf[pl.ds(k * 16, 16)]
    acc_vm[...] = acc
    pltpu.sync_copy(acc_vm, out_hbm.at[wid])


def _sc_lse_partials(idx_flat, lse):
    n_rows = idx_flat.shape[0]
    rows_per_w = n_rows // _NW
    n_dma = rows_per_w // 128
    idx3 = idx_flat.reshape(_NW, n_dma, 128)
    mesh = plsc.VectorSubcoreMesh(
        core_axis_name="c", subcore_axis_name="s",
        num_cores=_NC, num_subcores=_NS)
    return pl.kernel(
        functools.partial(_nll_body, rows_per_w=rows_per_w),
        out_type=jax.ShapeDtypeStruct((_NW, 16), jnp.float32),
        mesh=mesh,
        scratch_types=(
            pltpu.VMEM((n_dma, 128), jnp.int32),
            pltpu.VMEM((rows_per_w,), jnp.float32),
            pltpu.VMEM((16,), jnp.float32),
            pltpu.SemaphoreType.DMA,
        ),
    )(idx3, lse)


def _mean_body(lse_part_ref, tv_part_ref, loss_ref, *, n_rows):
    nll_sum = jnp.sum(lse_part_ref[...]) - jnp.sum(tv_part_ref[...])
    loss_ref[...] = jnp.full((1, 1), nll_sum / jnp.float32(n_rows),
                             jnp.float32)


def kernel(idx, targets, table):
    B, T = idx.shape
    N = B * T
    V, C = table.shape
    chunk = _CHUNK
    idx_flat = idx.reshape(N).astype(jnp.int32)
    tgt_flat = targets.reshape(N).astype(jnp.int32)
    grp = 2 * chunk
    tgt16 = jnp.pad(tgt_flat.reshape(_NW, N // (_NW * grp), 1, grp),
                    ((0, 0), (0, 0), (0, 0), (0, 16 - grp))
                    ).reshape(_NW, N // (_NW * grp), 16)

    logits, tv_partials = _sc_gather(idx_flat, tgt16, table)
    lse = _tc_lse(table)
    lse_partials = _sc_lse_partials(idx_flat, lse)
    loss = pl.pallas_call(
        functools.partial(_mean_body, n_rows=N),
        out_shape=jax.ShapeDtypeStruct((1, 1), jnp.float32),
    )(lse_partials, tv_partials)
    return (logits, loss[0, 0])
